# baseline (speedup 1.0000x reference)
import functools
import math

import jax
import jax.numpy as jnp
from jax import lax
from jax.experimental import pallas as pl
from jax.experimental.pallas import tpu as pltpu
from jax.experimental.pallas import tpu_sc as plsc

NC = 2
NS = 16
NW = NC * NS
L = 16
CHUNK = 128
NB = 4
IB = 20
FH = 64


def _degree_kernel(n_chunks, npad):
    cpt = n_chunks // NW
    rpt = npad // NS
    mesh = plsc.VectorSubcoreMesh(core_axis_name="c", subcore_axis_name="s")

    @functools.partial(
        pl.kernel,
        out_type=jax.ShapeDtypeStruct((NC * npad, L), jnp.float32),
        mesh=mesh,
        scratch_types=[
            pltpu.VMEM((cpt, 2, CHUNK), jnp.int32),
            pltpu.VMEM((CHUNK, L), jnp.float32),
            pltpu.VMEM((16, L), jnp.float32),
            pltpu.VMEM_SHARED((npad, L), jnp.float32),
            pltpu.SemaphoreType.DMA,
        ]
        + [pltpu.SemaphoreType.DMA for _ in range(NB)],
        name="sc_gcn_degree",
    )
    def deg_kernel(e_hbm, out_hbm, idx_v, ones_v, zero_v, acc_sh, sem_i,
                   *sem_s):
        cid = lax.axis_index("c")
        sid = lax.axis_index("s")
        wid = cid * NS + sid

        pltpu.async_copy(e_hbm.at[pl.ds(wid * cpt, cpt)], idx_v, sem_i)

        zeros16 = jnp.zeros((L,), jnp.float32)
        ones16 = jnp.ones((L,), jnp.float32)
        for i in range(CHUNK):
            ones_v[i, :] = ones16
        for i in range(16):
            zero_v[i, :] = zeros16

        def zero_body(k, carry):
            pltpu.sync_copy(zero_v, acc_sh.at[pl.ds(sid * rpt + k * 16, 16)])
            return carry

        lax.fori_loop(0, rpt // 16, zero_body, 0)
        pltpu.make_async_copy(e_hbm.at[pl.ds(wid * cpt, cpt)], idx_v,
                              sem_i).wait()
        plsc.subcore_barrier()

        def fire(b, k):
            pltpu.async_copy(ones_v, acc_sh.at[idx_v.at[k, 1]], sem_s[b],
                             add=True)

        def drain(b, k):
            pltpu.make_async_copy(ones_v, acc_sh.at[idx_v.at[k, 1]],
                                  sem_s[b]).wait()

        for b in range(NB):
            fire(b, b)

        def round_body(r, carry):
            for b in range(NB):
                drain(b, (r - 1) * NB + b)
                fire(b, r * NB + b)
            return carry

        lax.fori_loop(1, cpt // NB, round_body, 0)
        for b in range(NB):
            drain(b, (cpt // NB - 1) * NB + b)

        plsc.subcore_barrier()
        pltpu.sync_copy(
            acc_sh.at[pl.ds(sid * rpt, rpt)],
            out_hbm.at[pl.ds(cid * npad + sid * rpt, rpt)],
        )

    return deg_kernel


def _agg_kernel(n_chunks, npad):
    cpt = n_chunks // NS
    nblk = cpt // IB
    n_ir = IB // NB
    rpt = npad // NS
    mesh = plsc.VectorSubcoreMesh(core_axis_name="c", subcore_axis_name="s")

    @functools.partial(
        pl.kernel,
        out_type=jax.ShapeDtypeStruct((NC * npad, FH), jnp.float32),
        mesh=mesh,
        scratch_types=[
            pltpu.VMEM((2, IB, 2, CHUNK), jnp.int32),
            pltpu.VMEM((NB, CHUNK, FH), jnp.float32),
            pltpu.VMEM_SHARED((npad, FH), jnp.float32),
            pltpu.VMEM_SHARED((npad, FH), jnp.float32),
            pltpu.SemaphoreType.DMA,
            pltpu.SemaphoreType.DMA,
            pltpu.SemaphoreType.DMA,
        ]
        + [pltpu.SemaphoreType.DMA for _ in range(2 * NB)],
        compiler_params=pltpu.CompilerParams(use_tc_tiling_on_sc=False),
        name="sc_gcn_aggregate",
    )
    def agg(y_hbm, e_hbm, out_hbm, idx_v, rows_v, y_sh, acc_sh, *sems):
        sem_i = sems[:2]
        sem_y = sems[2]
        sem_g = sems[3:3 + NB]
        sem_s = sems[3 + NB:]
        cid = lax.axis_index("c")
        sid = lax.axis_index("s")

        def fire_idx(p, blk):
            pltpu.async_copy(e_hbm.at[pl.ds(sid * cpt + blk * IB, IB)],
                             idx_v.at[p], sem_i[p])

        def wait_idx(p, blk):
            pltpu.make_async_copy(e_hbm.at[pl.ds(sid * cpt + blk * IB, IB)],
                                  idx_v.at[p], sem_i[p]).wait()

        fire_idx(0, 0)
        pltpu.async_copy(y_hbm.at[cid].at[pl.ds(sid * rpt, rpt)],
                         y_sh.at[pl.ds(sid * rpt, rpt)], sem_y)
        pltpu.async_copy(y_hbm.at[cid].at[pl.ds(sid * rpt, rpt)],
                         acc_sh.at[pl.ds(sid * rpt, rpt)], sem_y)
        pltpu.make_async_copy(y_hbm.at[cid].at[pl.ds(sid * rpt, rpt)],
                              y_sh.at[pl.ds(sid * rpt, rpt)], sem_y).wait()
        pltpu.make_async_copy(y_hbm.at[cid].at[pl.ds(sid * rpt, rpt)],
                              acc_sh.at[pl.ds(sid * rpt, rpt)], sem_y).wait()
        wait_idx(0, 0)
        plsc.subcore_barrier()

        def _gather_src(b, p, k):
            if b < NB // 2:
                return y_hbm.at[cid].at[idx_v.at[p, k, 0]]
            return y_sh.at[idx_v.at[p, k, 0]]

        def fire_gather(b, p, k):
            pltpu.async_copy(_gather_src(b, p, k), rows_v.at[b], sem_g[b])

        def wait_gather(b, p, k):
            pltpu.make_async_copy(_gather_src(b, p, k), rows_v.at[b],
                                  sem_g[b]).wait()

        def fire_scatter(b, p, k):
            pltpu.async_copy(rows_v.at[b], acc_sh.at[idx_v.at[p, k, 1]],
                             sem_s[b], add=True)

        def wait_scatter(b, p, k):
            pltpu.make_async_copy(rows_v.at[b], acc_sh.at[idx_v.at[p, k, 1]],
                                  sem_s[b]).wait()

        for b in range(NB):
            fire_gather(b, 0, b)

        for blk in range(nblk):
            p = blk % 2
            pn = (blk + 1) % 2
            if blk + 1 < nblk:
                fire_idx(pn, blk + 1)

            def round_body(r, carry, p=p):
                base = r * NB
                for b in range(NB):
                    wait_gather(b, p, base + b)
                    fire_scatter(b, p, base + b)
                for b in range(NB):
                    wait_scatter(b, p, base + b)
                    fire_gather(b, p, base + NB + b)
                return carry

            lax.fori_loop(0, n_ir - 1, round_body, 0)
            base = (n_ir - 1) * NB
            for b in range(NB):
                wait_gather(b, p, base + b)
                fire_scatter(b, p, base + b)
            if blk + 1 < nblk:
                wait_idx(pn, blk + 1)
                for b in range(NB):
                    wait_scatter(b, p, base + b)
                    fire_gather(b, pn, b)
            else:
                for b in range(NB):
                    wait_scatter(b, p, base + b)

        plsc.subcore_barrier()
        pltpu.sync_copy(
            acc_sh.at[pl.ds(sid * rpt, rpt)],
            out_hbm.at[pl.ds(cid * npad + sid * rpt, rpt)],
        )

    return agg


def _dinv_from_partials(degp_ref, npad):
    deg = degp_ref[0:npad, 0:1] + degp_ref[npad:2 * npad, 0:1] + 1.0
    return lax.rsqrt(deg)


def _split(m, npad):
    return jnp.concatenate([m[:, 0:FH], m[:, FH:2 * FH]], axis=0)


def _tc_first(x_ref, w_ref, degp_ref, y_ref, *, npad):
    dinv = _dinv_from_partials(degp_ref, npad)
    xl = jnp.dot(x_ref[...], w_ref[...], preferred_element_type=jnp.float32)
    y_ref[...] = _split(xl * dinv, npad)


def _tc_mid(agg_ref, degp_ref, w_ref, b_ref, out_ref, *, npad):
    dinv = _dinv_from_partials(degp_ref, npad)
    s = agg_ref[...]
    full = jnp.concatenate([s[0:npad, :], s[npad:2 * npad, :]], axis=1)
    h = jnp.maximum(full * dinv + b_ref[...], 0.0)
    out_ref[...] = _split(
        jnp.dot(h, w_ref[...], preferred_element_type=jnp.float32) * dinv,
        npad,
    )


def _tc_last(agg_ref, degp_ref, b_ref, wfc_ref, bfc_ref, out_ref, *,
             npad):
    dinv = _dinv_from_partials(degp_ref, npad)
    s = agg_ref[...]
    full = jnp.concatenate([s[0:npad, :], s[npad:2 * npad, :]], axis=1)
    h = jnp.maximum(full * dinv + b_ref[...], 0.0)
    out_ref[...] = (
        jnp.dot(h, wfc_ref[...], preferred_element_type=jnp.float32)
        + bfc_ref[...]
    )


def kernel(x, edge_index, W1, b1, W2, b2, Wfc, bfc):
    n, nf = x.shape
    e = edge_index.shape[1]
    nclass = Wfc.shape[1]

    npad = ((n + 1 + 255) // 256) * 256
    echunk = math.lcm(NW * NB, NS * IB) * CHUNK
    ep = ((e + echunk - 1) // echunk) * echunk
    n_chunks = ep // CHUNK

    src = edge_index[0].astype(jnp.int32)
    dst = edge_index[1].astype(jnp.int32)
    pad_idx = jnp.full((ep - e,), n, dtype=jnp.int32)
    src_p = jnp.concatenate([src, pad_idx]).reshape(-1, 1, CHUNK)
    dst_p = jnp.concatenate([dst, pad_idx]).reshape(-1, 1, CHUNK)
    edges = jnp.concatenate([src_p, dst_p], axis=1)
    x_p = jnp.zeros((npad, nf), jnp.float32).at[:n, :].set(x)

    degp = _degree_kernel(n_chunks, npad)(edges)

    tc1 = pl.pallas_call(
        functools.partial(_tc_first, npad=npad),
        out_shape=jax.ShapeDtypeStruct((NC * npad, FH), jnp.float32),
        name="tc_gcn_xw_scale",
    )
    y1 = tc1(x_p, W1, degp)

    agg = _agg_kernel(n_chunks, npad)
    a1 = agg(y1.reshape(NC, npad, FH), edges)

    tc2 = pl.pallas_call(
        functools.partial(_tc_mid, npad=npad),
        out_shape=jax.ShapeDtypeStruct((NC * npad, FH), jnp.float32),
        name="tc_gcn_layer2",
    )
    y2 = tc2(a1, degp, W2, b1)

    a2 = agg(y2.reshape(NC, npad, FH), edges)

    tc3 = pl.pallas_call(
        functools.partial(_tc_last, npad=npad),
        out_shape=jax.ShapeDtypeStruct((npad, nclass), jnp.float32),
        name="tc_gcn_head",
    )
    out = tc3(a2, degp, b2, Wfc, bfc)
    return out[:n]


# speedup vs baseline: 1.1640x; 1.1640x over previous
import functools
import math

import jax
import jax.numpy as jnp
from jax import lax
from jax.experimental import pallas as pl
from jax.experimental.pallas import tpu as pltpu
from jax.experimental.pallas import tpu_sc as plsc

NC = 2
NS = 16
NW = NC * NS
L = 16
CHUNK = 128
NB = 4
IB = 20
FH = 64


def _degree_kernel(n_chunks, npad):
    cpt = n_chunks // NW
    rpt = npad // NS
    mesh = plsc.VectorSubcoreMesh(core_axis_name="c", subcore_axis_name="s")

    @functools.partial(
        pl.kernel,
        out_type=jax.ShapeDtypeStruct((NC * npad, L), jnp.float32),
        mesh=mesh,
        scratch_types=[
            pltpu.VMEM((cpt, 2, CHUNK), jnp.int32),
            pltpu.VMEM((CHUNK, L), jnp.float32),
            pltpu.VMEM((16, L), jnp.float32),
            pltpu.VMEM_SHARED((npad, L), jnp.float32),
            pltpu.SemaphoreType.DMA,
        ]
        + [pltpu.SemaphoreType.DMA for _ in range(NB)],
        name="sc_gcn_degree",
    )
    def deg_kernel(e_hbm, out_hbm, idx_v, ones_v, zero_v, acc_sh, sem_i,
                   *sem_s):
        cid = lax.axis_index("c")
        sid = lax.axis_index("s")
        wid = cid * NS + sid

        pltpu.async_copy(e_hbm.at[pl.ds(wid * cpt, cpt)], idx_v, sem_i)

        zeros16 = jnp.zeros((L,), jnp.float32)
        ones16 = jnp.ones((L,), jnp.float32)
        for i in range(CHUNK):
            ones_v[i, :] = ones16
        for i in range(16):
            zero_v[i, :] = zeros16

        def zero_body(k, carry):
            pltpu.sync_copy(zero_v, acc_sh.at[pl.ds(sid * rpt + k * 16, 16)])
            return carry

        lax.fori_loop(0, rpt // 16, zero_body, 0)
        pltpu.make_async_copy(e_hbm.at[pl.ds(wid * cpt, cpt)], idx_v,
                              sem_i).wait()
        plsc.subcore_barrier()

        def fire(b, k):
            pltpu.async_copy(ones_v, acc_sh.at[idx_v.at[k, 1]], sem_s[b],
                             add=True)

        def drain(b, k):
            pltpu.make_async_copy(ones_v, acc_sh.at[idx_v.at[k, 1]],
                                  sem_s[b]).wait()

        for b in range(NB):
            fire(b, b)

        def round_body(r, carry):
            for b in range(NB):
                drain(b, (r - 1) * NB + b)
                fire(b, r * NB + b)
            return carry

        lax.fori_loop(1, cpt // NB, round_body, 0)
        for b in range(NB):
            drain(b, (cpt // NB - 1) * NB + b)

        plsc.subcore_barrier()
        pltpu.sync_copy(
            acc_sh.at[pl.ds(sid * rpt, rpt)],
            out_hbm.at[pl.ds(cid * npad + sid * rpt, rpt)],
        )

    return deg_kernel


def _agg_kernel(n_chunks, npad):
    cpt = n_chunks // NS
    nblk = cpt // IB
    n_ir = IB // NB
    rpt = npad // NS
    mesh = plsc.VectorSubcoreMesh(core_axis_name="c", subcore_axis_name="s")

    @functools.partial(
        pl.kernel,
        out_type=jax.ShapeDtypeStruct((NC * npad, FH), jnp.float32),
        mesh=mesh,
        scratch_types=[
            pltpu.VMEM((2, IB, 2, CHUNK), jnp.int32),
            pltpu.VMEM((NB, CHUNK, FH), jnp.float32),
            pltpu.VMEM_SHARED((npad, FH), jnp.float32),
            pltpu.VMEM_SHARED((npad, FH), jnp.float32),
            pltpu.SemaphoreType.DMA,
            pltpu.SemaphoreType.DMA,
            pltpu.SemaphoreType.DMA,
        ]
        + [pltpu.SemaphoreType.DMA for _ in range(2 * NB)],
        compiler_params=pltpu.CompilerParams(use_tc_tiling_on_sc=False),
        name="sc_gcn_aggregate",
    )
    def agg(y_hbm, e_hbm, out_hbm, idx_v, rows_v, y_sh, acc_sh, *sems):
        sem_i = sems[:2]
        sem_y = sems[2]
        sem_g = sems[3:3 + NB]
        sem_s = sems[3 + NB:]
        cid = lax.axis_index("c")
        sid = lax.axis_index("s")

        def fire_idx(p, blk):
            pltpu.async_copy(e_hbm.at[pl.ds(sid * cpt + blk * IB, IB)],
                             idx_v.at[p], sem_i[p])

        def wait_idx(p, blk):
            pltpu.make_async_copy(e_hbm.at[pl.ds(sid * cpt + blk * IB, IB)],
                                  idx_v.at[p], sem_i[p]).wait()

        fire_idx(0, 0)
        pltpu.async_copy(y_hbm.at[pl.ds(cid * npad + sid * rpt, rpt)],
                         y_sh.at[pl.ds(sid * rpt, rpt)], sem_y)
        pltpu.async_copy(y_hbm.at[pl.ds(cid * npad + sid * rpt, rpt)],
                         acc_sh.at[pl.ds(sid * rpt, rpt)], sem_y)
        pltpu.make_async_copy(y_hbm.at[pl.ds(cid * npad + sid * rpt, rpt)],
                              y_sh.at[pl.ds(sid * rpt, rpt)], sem_y).wait()
        pltpu.make_async_copy(y_hbm.at[pl.ds(cid * npad + sid * rpt, rpt)],
                              acc_sh.at[pl.ds(sid * rpt, rpt)], sem_y).wait()
        wait_idx(0, 0)
        plsc.subcore_barrier()

        def fire_gather(b, p, k):
            pltpu.async_copy(y_sh.at[idx_v.at[p, k, 0]], rows_v.at[b],
                             sem_g[b])

        def wait_gather(b, p, k):
            pltpu.make_async_copy(y_sh.at[idx_v.at[p, k, 0]], rows_v.at[b],
                                  sem_g[b]).wait()

        def fire_scatter(b, p, k):
            pltpu.async_copy(rows_v.at[b], acc_sh.at[idx_v.at[p, k, 1]],
                             sem_s[b], add=True)

        def wait_scatter(b, p, k):
            pltpu.make_async_copy(rows_v.at[b], acc_sh.at[idx_v.at[p, k, 1]],
                                  sem_s[b]).wait()

        for b in range(NB):
            fire_gather(b, 0, b)

        for blk in range(nblk):
            p = blk % 2
            pn = (blk + 1) % 2
            if blk + 1 < nblk:
                fire_idx(pn, blk + 1)

            def round_body(r, carry, p=p):
                base = r * NB
                for b in range(NB):
                    wait_gather(b, p, base + b)
                    fire_scatter(b, p, base + b)
                for b in range(NB):
                    wait_scatter(b, p, base + b)
                    fire_gather(b, p, base + NB + b)
                return carry

            lax.fori_loop(0, n_ir - 1, round_body, 0)
            base = (n_ir - 1) * NB
            for b in range(NB):
                wait_gather(b, p, base + b)
                fire_scatter(b, p, base + b)
            if blk + 1 < nblk:
                wait_idx(pn, blk + 1)
                for b in range(NB):
                    wait_scatter(b, p, base + b)
                    fire_gather(b, pn, b)
            else:
                for b in range(NB):
                    wait_scatter(b, p, base + b)

        plsc.subcore_barrier()
        pltpu.sync_copy(
            acc_sh.at[pl.ds(sid * rpt, rpt)],
            out_hbm.at[pl.ds(cid * npad + sid * rpt, rpt)],
        )

    return agg


def _dinv_from_partials(degp_ref, npad):
    deg = degp_ref[0:npad, 0:1] + degp_ref[npad:2 * npad, 0:1] + 1.0
    return lax.rsqrt(deg)


def _split(m, npad):
    return jnp.concatenate([m[:, 0:FH], m[:, FH:2 * FH]], axis=0)


def _tc_first(x_ref, w_ref, degp_ref, y_ref, *, npad):
    dinv = _dinv_from_partials(degp_ref, npad)
    xl = jnp.dot(x_ref[...], w_ref[...], preferred_element_type=jnp.float32)
    y_ref[...] = _split(xl * dinv, npad)


def _tc_mid(agg_ref, degp_ref, w_ref, b_ref, out_ref, *, npad):
    dinv = _dinv_from_partials(degp_ref, npad)
    s = agg_ref[...]
    full = jnp.concatenate([s[0:npad, :], s[npad:2 * npad, :]], axis=1)
    h = jnp.maximum(full * dinv + b_ref[...], 0.0)
    out_ref[...] = _split(
        jnp.dot(h, w_ref[...], preferred_element_type=jnp.float32) * dinv,
        npad,
    )


def _tc_last(agg_ref, degp_ref, b_ref, wfc_ref, bfc_ref, out_ref, *,
             npad):
    dinv = _dinv_from_partials(degp_ref, npad)
    s = agg_ref[...]
    full = jnp.concatenate([s[0:npad, :], s[npad:2 * npad, :]], axis=1)
    h = jnp.maximum(full * dinv + b_ref[...], 0.0)
    out_ref[...] = (
        jnp.dot(h, wfc_ref[...], preferred_element_type=jnp.float32)
        + bfc_ref[...]
    )


def kernel(x, edge_index, W1, b1, W2, b2, Wfc, bfc):
    n, nf = x.shape
    e = edge_index.shape[1]
    nclass = Wfc.shape[1]

    npad = ((n + 1 + 255) // 256) * 256
    echunk = math.lcm(NW * NB, NS * IB) * CHUNK
    ep = ((e + echunk - 1) // echunk) * echunk
    n_chunks = ep // CHUNK

    src = edge_index[0].astype(jnp.int32)
    dst = edge_index[1].astype(jnp.int32)
    pad_idx = jnp.full((ep - e,), n, dtype=jnp.int32)
    src_p = jnp.concatenate([src, pad_idx]).reshape(-1, 1, CHUNK)
    dst_p = jnp.concatenate([dst, pad_idx]).reshape(-1, 1, CHUNK)
    edges = jnp.concatenate([src_p, dst_p], axis=1)
    x_p = jnp.zeros((npad, nf), jnp.float32).at[:n, :].set(x)

    degp = _degree_kernel(n_chunks, npad)(edges)

    tc1 = pl.pallas_call(
        functools.partial(_tc_first, npad=npad),
        out_shape=jax.ShapeDtypeStruct((NC * npad, FH), jnp.float32),
        name="tc_gcn_xw_scale",
    )
    y1 = tc1(x_p, W1, degp)

    agg = _agg_kernel(n_chunks, npad)
    a1 = agg(y1, edges)

    tc2 = pl.pallas_call(
        functools.partial(_tc_mid, npad=npad),
        out_shape=jax.ShapeDtypeStruct((NC * npad, FH), jnp.float32),
        name="tc_gcn_layer2",
    )
    y2 = tc2(a1, degp, W2, b1)

    a2 = agg(y2, edges)

    tc3 = pl.pallas_call(
        functools.partial(_tc_last, npad=npad),
        out_shape=jax.ShapeDtypeStruct((npad, nclass), jnp.float32),
        name="tc_gcn_head",
    )
    out = tc3(a2, degp, b2, Wfc, bfc)
    return out[:n]
